# baseline (speedup 1.0000x reference)
import jax
import jax.numpy as jnp
from jax import lax
from jax.experimental import pallas as pl
from jax.experimental.pallas import tpu as pltpu
from jax.experimental.pallas import tpu_sc as plsc

B = 16384
D = 16
NC = 2
NS = 16
L = 16
NW = NC * NS
BPW = B // NW
CHUNK = 15
NCHUNK = -(-BPW // CHUNK)
PAD = 544


def _cmf_body(uid_hbm, iid_hbm, utab_hbm, itab_hbm, out_hbm,
              uid_v, iid_v, ubuf_v, ibuf_v, out_v,
              sem_u0, sem_i0, sem_u1, sem_i1):
    wid = lax.axis_index("s") * NC + lax.axis_index("c")
    base = wid * BPW

    lane = lax.iota(jnp.int32, L)
    zeros = jnp.zeros((L,), jnp.int32)
    uid_v[pl.ds(BPW, L)] = zeros
    uid_v[pl.ds(PAD - L, L)] = zeros
    iid_v[pl.ds(BPW, L)] = zeros
    iid_v[pl.ds(PAD - L, L)] = zeros
    pltpu.sync_copy(uid_hbm.at[pl.ds(base, BPW)], uid_v.at[pl.ds(0, BPW)])
    pltpu.sync_copy(iid_hbm.at[pl.ds(base, BPW)], iid_v.at[pl.ds(0, BPW)])

    def chunk_ids(c):
        idx = c * CHUNK + lane
        uvec = plsc.load_gather(uid_v, [idx])
        ivec = plsc.load_gather(iid_v, [idx])
        return uvec, ivec

    def issue(c, ubuf, ibuf, sem_u, sem_i):
        uvec, ivec = chunk_ids(c)
        cu = jnp.right_shift(uvec, 7) * 128
        ci = jnp.right_shift(ivec, 7) * 128
        for j in range(CHUNK):
            cuj = pl.multiple_of(jnp.sum(jnp.where(lane == j, cu, 0)), 128)
            cij = pl.multiple_of(jnp.sum(jnp.where(lane == j, ci, 0)), 128)
            pltpu.async_copy(utab_hbm.at[:, pl.ds(cuj, 128)], ubuf.at[j], sem_u)
            pltpu.async_copy(itab_hbm.at[:, pl.ds(cij, 128)], ibuf.at[j], sem_i)

    def drain(ubuf, ibuf, sem_u, sem_i):
        for j in range(CHUNK):
            pltpu.make_async_copy(
                utab_hbm.at[:, pl.ds(0, 128)], ubuf.at[j], sem_u).wait()
            pltpu.make_async_copy(
                itab_hbm.at[:, pl.ds(0, 128)], ibuf.at[j], sem_i).wait()

    row = jnp.minimum(lane, CHUNK - 1)

    def compute(c, ubuf, ibuf):
        uvec, ivec = chunk_ids(c)
        lu = jnp.bitwise_and(uvec, 127)
        li = jnp.bitwise_and(ivec, 127)
        acc = jnp.zeros((L,), jnp.float32)
        for d in range(D):
            dsplat = jnp.full((L,), d, jnp.int32)
            u = plsc.load_gather(ubuf, [row, dsplat, lu])
            it = plsc.load_gather(ibuf, [row, dsplat, li])
            acc = acc + u * it
        res = 1.0 / (1.0 + jnp.exp(-acc))
        plsc.store_scatter(out_v, [c * CHUNK + lane], res)

    issue(0, ubuf_v.at[0], ibuf_v.at[0], sem_u0, sem_i0)

    def chunk_body(c, carry):
        @pl.when(c % 2 == 0)
        def _even():
            @pl.when(c < NCHUNK - 1)
            def _pf():
                issue(c + 1, ubuf_v.at[1], ibuf_v.at[1], sem_u1, sem_i1)
            drain(ubuf_v.at[0], ibuf_v.at[0], sem_u0, sem_i0)
            compute(c, ubuf_v.at[0], ibuf_v.at[0])

        @pl.when(c % 2 == 1)
        def _odd():
            @pl.when(c < NCHUNK - 1)
            def _pf():
                issue(c + 1, ubuf_v.at[0], ibuf_v.at[0], sem_u0, sem_i0)
            drain(ubuf_v.at[1], ibuf_v.at[1], sem_u1, sem_i1)
            compute(c, ubuf_v.at[1], ibuf_v.at[1])

        return carry

    lax.fori_loop(0, NCHUNK, chunk_body, 0)
    pltpu.sync_copy(out_v.at[pl.ds(0, BPW)], out_hbm.at[pl.ds(base, BPW)])


def kernel(user_ids, item_ids, source_user, source_item):
    mesh = plsc.VectorSubcoreMesh(
        core_axis_name="c", subcore_axis_name="s",
        num_cores=NC, num_subcores=NS)
    k = pl.kernel(
        _cmf_body,
        out_type=jax.ShapeDtypeStruct((B,), jnp.float32),
        mesh=mesh,
        compiler_params=pltpu.CompilerParams(
            needs_layout_passes=False, use_tc_tiling_on_sc=True),
        scratch_types=[
            pltpu.VMEM((PAD,), jnp.int32),
            pltpu.VMEM((PAD,), jnp.int32),
            pltpu.VMEM((2, CHUNK, D, 128), jnp.float32),
            pltpu.VMEM((2, CHUNK, D, 128), jnp.float32),
            pltpu.VMEM((PAD,), jnp.float32),
            pltpu.SemaphoreType.DMA,
            pltpu.SemaphoreType.DMA,
            pltpu.SemaphoreType.DMA,
            pltpu.SemaphoreType.DMA,
        ],
    )
    return k(user_ids.astype(jnp.int32), item_ids.astype(jnp.int32),
             source_user.T, source_item.T)
